# Initial kernel scaffold; baseline (speedup 1.0000x reference)
import jax
import jax.numpy as jnp
from jax.experimental import pallas as pl


def kernel(x, edge_index, batch, W, a_src, a_dst, W2, b2):
    raise NotImplementedError("write your pallas kernel here")


# speedup vs baseline: 66.2266x; 66.2266x over previous
import functools

import jax
import jax.numpy as jnp
from jax import lax
from jax.experimental import pallas as pl
from jax.experimental.pallas import tpu as pltpu
from jax.experimental.pallas import tpu_sc as plsc

N = 10000
E = 320000
D = 128
H = 4
DH = 32
G = 64
C = 10

ACC_W = 144
K = 128
NW = 32
EPW = E // NW
NFULL = EPW // K
TAIL = EPW - NFULL * K
N_PAD = 10240
RPT = N_PAD // 16


def _tc_prep(x, W, Am):
    blk = 1000
    nb = N // blk

    def body(x_ref, w_ref, am_ref, h_ref, a_ref):
        hb = jnp.dot(x_ref[...], w_ref[...], preferred_element_type=jnp.float32)
        h_ref[...] = hb
        a_ref[...] = jnp.dot(hb, am_ref[...], preferred_element_type=jnp.float32)

    return pl.pallas_call(
        body,
        grid=(nb,),
        in_specs=[
            pl.BlockSpec((blk, D), lambda i: (i, 0)),
            pl.BlockSpec((D, D), lambda i: (0, 0)),
            pl.BlockSpec((D, 8), lambda i: (0, 0)),
        ],
        out_specs=[
            pl.BlockSpec((blk, D), lambda i: (i, 0)),
            pl.BlockSpec((blk, 8), lambda i: (i, 0)),
        ],
        out_shape=[
            jax.ShapeDtypeStruct((N, D), jnp.float32),
            jax.ShapeDtypeStruct((N, 8), jnp.float32),
        ],
    )(x, W, Am)


def _sc_edges(h, a8, src, dst):
    mesh = plsc.VectorSubcoreMesh(core_axis_name="c", subcore_axis_name="s")

    @functools.partial(
        pl.kernel,
        mesh=mesh,
        out_type=[
            jax.ShapeDtypeStruct((2, N_PAD, D), jnp.float32),
            jax.ShapeDtypeStruct((2, N_PAD, 16), jnp.float32),
        ],
        compiler_params=pltpu.CompilerParams(use_tc_tiling_on_sc=False,
                                             needs_layout_passes=False),
        scratch_types=[
            pltpu.VMEM((K,), jnp.int32),
            pltpu.VMEM((K,), jnp.int32),
            pltpu.VMEM((16,), jnp.int32),
            pltpu.VMEM((16,), jnp.int32),
            pltpu.VMEM((K, D), jnp.float32),
            pltpu.VMEM((K, 8), jnp.float32),
            pltpu.VMEM((K, 8), jnp.float32),
            pltpu.VMEM((4, K), jnp.float32),
            pltpu.VMEM((K, 16), jnp.float32),
            pltpu.VMEM_SHARED((N_PAD, D), jnp.float32),
            pltpu.VMEM_SHARED((N_PAD, 16), jnp.float32),
            pltpu.SemaphoreType.DMA,
            pltpu.SemaphoreType.DMA,
        ],
    )
    def k(h_hbm, a8_hbm, src_hbm, dst_hbm, msg_hbm, den_hbm,
          ixs, ixd, ixs_t, ixd_t, rows, ra, rb, wbuf, wrow,
          macc, dacc, sem, sem2):
        cid = lax.axis_index("c")
        sid = lax.axis_index("s")
        wid = cid * 16 + sid
        ebase = wid * EPW
        lane = lax.iota(jnp.int32, 16)
        zv = jnp.zeros((16,), jnp.float32)

        def zrow(r, _):
            for c8 in range(D // 16):
                rows[r, pl.ds(c8 * 16, 16)] = zv
            wrow[r, pl.ds(0, 16)] = zv
            return 0
        lax.fori_loop(0, K, zrow, 0)
        rbase = sid * RPT
        for j in range(RPT // K):
            pltpu.sync_copy(rows, macc.at[pl.ds(rbase + j * K, K)])
            pltpu.sync_copy(wrow, dacc.at[pl.ds(rbase + j * K, K)])
        plsc.subcore_barrier()

        def do_chunk(base, ixs_r, ixd_r, nk):
            pltpu.sync_copy(src_hbm.at[pl.ds(base, nk)], ixs_r)
            pltpu.sync_copy(dst_hbm.at[pl.ds(base, nk)], ixd_r)
            rows_r = rows if nk == K else rows.at[pl.ds(0, nk)]
            ra_r = ra if nk == K else ra.at[pl.ds(0, nk)]
            rb_r = rb if nk == K else rb.at[pl.ds(0, nk)]
            ca = pltpu.async_copy(a8_hbm.at[ixs_r], ra_r, sem2)
            cb = pltpu.async_copy(a8_hbm.at[ixd_r], rb_r, sem2)
            ch = pltpu.async_copy(h_hbm.at[ixs_r], rows_r, sem)
            ca.wait()
            cb.wait()
            for g in range(nk // 16):
                rix = lane + (g * 16)
                for hd in range(H):
                    e = (plsc.load_gather(ra, [rix, jnp.full((16,), hd,
                                                            jnp.int32)]) +
                         plsc.load_gather(rb, [rix, jnp.full((16,), 4 + hd,
                                                            jnp.int32)]))
                    e = jnp.where(e >= 0.0, e, e * jnp.float32(0.2))
                    w = jnp.exp(e)
                    wbuf[hd, pl.ds(g * 16, 16)] = w
                    plsc.store_scatter(
                        wrow, [rix, jnp.full((16,), hd, jnp.int32)], w)
            ch.wait()
            def edge_body(e2, _):
                e2v = jnp.full((16,), e2, jnp.int32)
                for hd in range(H):
                    wb = plsc.load_gather(
                        wbuf, [jnp.full((16,), hd, jnp.int32), e2v])
                    for half in range(2):
                        off = hd * DH + half * 16
                        rows[e2, pl.ds(off, 16)] = (
                            rows[e2, pl.ds(off, 16)] * wb)
                return 0
            lax.fori_loop(0, nk, edge_body, 0)
            pltpu.sync_copy(rows_r, macc.at[ixd_r], add=True)
            wrow_r = wrow if nk == K else wrow.at[pl.ds(0, nk)]
            pltpu.sync_copy(wrow_r, dacc.at[ixd_r], add=True)

        def chunk_body(i, _):
            do_chunk(ebase + i * K, ixs, ixd, K)
            return 0
        lax.fori_loop(0, NFULL, chunk_body, 0)
        if TAIL:
            do_chunk(ebase + NFULL * K, ixs_t, ixd_t, TAIL)

        plsc.subcore_barrier()
        pltpu.sync_copy(macc.at[pl.ds(rbase, RPT)],
                        msg_hbm.at[cid, pl.ds(rbase, RPT)])
        pltpu.sync_copy(dacc.at[pl.ds(rbase, RPT)],
                        den_hbm.at[cid, pl.ds(rbase, RPT)])

    return k(h, a8, src, dst)


def _tc_post(m0, m1, d0, d1, batch3, W2, b2):
    blk = 1000
    nb = N // blk

    def body(m0_ref, m1_ref, d0_ref, d1_ref, b_ref, w2_ref, b2_ref,
             out_ref, gsum, gcnt):
        i = pl.program_id(0)

        @pl.when(i == 0)
        def _():
            gsum[...] = jnp.zeros((G, D), jnp.float32)
            gcnt[...] = jnp.zeros((G, D), jnp.float32)

        msg = m0_ref[...] + m1_ref[...]
        dsum = d0_ref[...] + d1_ref[...]
        den = dsum[:, 0:H] + jnp.float32(1e-16)
        expand = (lax.broadcasted_iota(jnp.int32, (H, D), 1) // DH ==
                  lax.broadcasted_iota(jnp.int32, (H, D), 0)
                  ).astype(jnp.float32)
        denf = jnp.dot(1.0 / den, expand, preferred_element_type=jnp.float32)
        node = msg * denf
        node = jnp.where(node > 0.0, node,
                         jnp.exp(jnp.minimum(node, 0.0)) - 1.0)
        bids = b_ref[0]
        gid = lax.broadcasted_iota(jnp.int32, (G, blk), 0)
        oh = (bids == gid).astype(jnp.float32)
        gsum[...] += jnp.dot(oh, node, preferred_element_type=jnp.float32)
        gcnt[...] += jnp.dot(oh, jnp.ones((blk, D), jnp.float32),
                             preferred_element_type=jnp.float32)

        @pl.when(i == nb - 1)
        def _():
            gr = gsum[...] / jnp.maximum(gcnt[...], 1.0)
            out_ref[...] = (jnp.dot(gr, w2_ref[...],
                                    preferred_element_type=jnp.float32)
                            + b2_ref[...])

    return pl.pallas_call(
        body,
        grid=(nb,),
        in_specs=[
            pl.BlockSpec((blk, D), lambda i: (i, 0)),
            pl.BlockSpec((blk, D), lambda i: (i, 0)),
            pl.BlockSpec((blk, 16), lambda i: (i, 0)),
            pl.BlockSpec((blk, 16), lambda i: (i, 0)),
            pl.BlockSpec((1, 1, blk), lambda i: (i, 0, 0)),
            pl.BlockSpec((D, C), lambda i: (0, 0)),
            pl.BlockSpec((1, C), lambda i: (0, 0)),
        ],
        out_specs=pl.BlockSpec((G, C), lambda i: (0, 0)),
        out_shape=jax.ShapeDtypeStruct((G, C), jnp.float32),
        scratch_shapes=[
            pltpu.VMEM((G, D), jnp.float32),
            pltpu.VMEM((G, D), jnp.float32),
        ],
    )(m0, m1, d0, d1, batch3, W2, b2)


def kernel(x, edge_index, batch, W, a_src, a_dst, W2, b2):
    Am = jnp.zeros((D, 2 * H), jnp.float32)
    for hd in range(H):
        Am = Am.at[hd * DH:(hd + 1) * DH, hd].set(a_src[hd])
        Am = Am.at[hd * DH:(hd + 1) * DH, H + hd].set(a_dst[hd])

    h, a8 = _tc_prep(x, W, Am)
    src = edge_index[0]
    dst = edge_index[1]

    msg, den = _sc_edges(h, a8, src, dst)

    batch3 = batch.reshape(N // 1000, 1, 1000)
    return _tc_post(msg[0], msg[1], den[0], den[1], batch3, W2,
                    b2.reshape(1, C))


# speedup vs baseline: 75.5536x; 1.1408x over previous
import functools

import jax
import jax.numpy as jnp
from jax import lax
from jax.experimental import pallas as pl
from jax.experimental.pallas import tpu as pltpu
from jax.experimental.pallas import tpu_sc as plsc

N = 10000
E = 320000
D = 128
H = 4
DH = 32
G = 64
C = 10

K = 96
NW = 32
EPW = E // NW
NFULL = EPW // K
TAIL = EPW - NFULL * K
N_PAD = 10240
RPT = N_PAD // 16


def _tc_prep(x, W, Am):
    blk = 1000
    nb = N // blk

    def body(x_ref, w_ref, am_ref, h_ref, a_ref):
        hb = jnp.dot(x_ref[...], w_ref[...], preferred_element_type=jnp.float32)
        h_ref[...] = hb
        a_ref[...] = jnp.dot(hb, am_ref[...], preferred_element_type=jnp.float32)

    return pl.pallas_call(
        body,
        grid=(nb,),
        in_specs=[
            pl.BlockSpec((blk, D), lambda i: (i, 0)),
            pl.BlockSpec((D, D), lambda i: (0, 0)),
            pl.BlockSpec((D, 8), lambda i: (0, 0)),
        ],
        out_specs=[
            pl.BlockSpec((blk, D), lambda i: (i, 0)),
            pl.BlockSpec((blk, 8), lambda i: (i, 0)),
        ],
        out_shape=[
            jax.ShapeDtypeStruct((N, D), jnp.float32),
            jax.ShapeDtypeStruct((N, 8), jnp.float32),
        ],
    )(x, W, Am)


def _sc_edges(h, a8, src, dst):
    mesh = plsc.VectorSubcoreMesh(core_axis_name="c", subcore_axis_name="s")

    @functools.partial(
        pl.kernel,
        mesh=mesh,
        out_type=[
            jax.ShapeDtypeStruct((2, N_PAD, D), jnp.float32),
            jax.ShapeDtypeStruct((2, N_PAD, 16), jnp.float32),
        ],
        compiler_params=pltpu.CompilerParams(use_tc_tiling_on_sc=False,
                                             needs_layout_passes=False),
        scratch_types=[
            pltpu.VMEM((K,), jnp.int32),
            pltpu.VMEM((K,), jnp.int32),
            pltpu.VMEM((K,), jnp.int32),
            pltpu.VMEM((K,), jnp.int32),
            pltpu.VMEM((16,), jnp.int32),
            pltpu.VMEM((16,), jnp.int32),
            pltpu.VMEM((K, D), jnp.float32),
            pltpu.VMEM((K, D), jnp.float32),
            pltpu.VMEM((K, 8), jnp.float32),
            pltpu.VMEM((K, 8), jnp.float32),
            pltpu.VMEM((K, 8), jnp.float32),
            pltpu.VMEM((K, 8), jnp.float32),
            pltpu.VMEM((4, K), jnp.float32),
            pltpu.VMEM((K, 16), jnp.float32),
            pltpu.VMEM((K, 16), jnp.float32),
            pltpu.VMEM_SHARED((N_PAD, D), jnp.float32),
            pltpu.VMEM_SHARED((N_PAD, 16), jnp.float32),
            pltpu.SemaphoreType.DMA,
            pltpu.SemaphoreType.DMA,
            pltpu.SemaphoreType.DMA,
            pltpu.SemaphoreType.DMA,
        ],
    )
    def k(h_hbm, a8_hbm, src_hbm, dst_hbm, msg_hbm, den_hbm,
          ixs0, ixs1, ixd0, ixd1, ixs_t, ixd_t, rows0, rows1,
          ra0, ra1, rb0, rb1, wbuf, wrow0, wrow1,
          macc, dacc, gsem0, gsem1, ssem0, ssem1):
        ixs = (ixs0, ixs1)
        ixd = (ixd0, ixd1)
        rows = (rows0, rows1)
        ra = (ra0, ra1)
        rb = (rb0, rb1)
        wrow = (wrow0, wrow1)
        gsem = (gsem0, gsem1)
        ssem = (ssem0, ssem1)

        cid = lax.axis_index("c")
        sid = lax.axis_index("s")
        wid = cid * 16 + sid
        ebase = wid * EPW
        lane = lax.iota(jnp.int32, 16)
        zv = jnp.zeros((16,), jnp.float32)

        def zrow(r, _):
            for c8 in range(D // 16):
                rows0[r, pl.ds(c8 * 16, 16)] = zv
            wrow0[r, pl.ds(0, 16)] = zv
            wrow1[r, pl.ds(0, 16)] = zv
            return 0
        lax.fori_loop(0, K, zrow, 0)
        rbase = sid * RPT
        nzf = RPT // K
        for j in range(nzf):
            pltpu.sync_copy(rows0, macc.at[pl.ds(rbase + j * K, K)])
            pltpu.sync_copy(wrow0, dacc.at[pl.ds(rbase + j * K, K)])
        zr = RPT - nzf * K
        if zr:
            pltpu.sync_copy(rows0.at[pl.ds(0, zr)],
                            macc.at[pl.ds(rbase + nzf * K, zr)])
            pltpu.sync_copy(wrow0.at[pl.ds(0, zr)],
                            dacc.at[pl.ds(rbase + nzf * K, zr)])
        plsc.subcore_barrier()

        def idx_load(c, p):
            base = ebase + c * K
            pltpu.sync_copy(src_hbm.at[pl.ds(base, K)], ixs[p])
            pltpu.sync_copy(dst_hbm.at[pl.ds(base, K)], ixd[p])

        def gathers_start(p):
            pltpu.async_copy(h_hbm.at[ixs[p]], rows[p], gsem[p])
            pltpu.async_copy(a8_hbm.at[ixs[p]], ra[p], gsem[p])
            pltpu.async_copy(a8_hbm.at[ixd[p]], rb[p], gsem[p])

        def gathers_wait(p):
            pltpu.make_async_copy(h_hbm.at[ixs[p]], rows[p], gsem[p]).wait()
            pltpu.make_async_copy(a8_hbm.at[ixs[p]], ra[p], gsem[p]).wait()
            pltpu.make_async_copy(a8_hbm.at[ixd[p]], rb[p], gsem[p]).wait()

        def scat_start(p):
            pltpu.async_copy(rows[p], macc.at[ixd[p]], ssem[p], add=True)
            pltpu.async_copy(wrow[p], dacc.at[ixd[p]], ssem[p], add=True)

        def scat_wait(p):
            pltpu.make_async_copy(rows[p], macc.at[ixd[p]], ssem[p]).wait()
            pltpu.make_async_copy(wrow[p], dacc.at[ixd[p]], ssem[p]).wait()

        def compute(p, nk):
            for g in range(nk // 16):
                rix = lane + (g * 16)
                for hd in range(H):
                    e = (plsc.load_gather(ra[p], [rix, jnp.full((16,), hd,
                                                                jnp.int32)]) +
                         plsc.load_gather(rb[p], [rix, jnp.full((16,), 4 + hd,
                                                                jnp.int32)]))
                    e = jnp.where(e >= 0.0, e, e * jnp.float32(0.2))
                    w = jnp.exp(e)
                    wbuf[hd, pl.ds(g * 16, 16)] = w
                    plsc.store_scatter(
                        wrow[p], [rix, jnp.full((16,), hd, jnp.int32)], w)
            def edge_body(e2, _):
                e2v = jnp.full((16,), e2, jnp.int32)
                for hd in range(H):
                    wb = plsc.load_gather(
                        wbuf, [jnp.full((16,), hd, jnp.int32), e2v])
                    for half in range(2):
                        off = hd * DH + half * 16
                        rows[p][e2, pl.ds(off, 16)] = (
                            rows[p][e2, pl.ds(off, 16)] * wb)
                return 0
            lax.fori_loop(0, nk, edge_body, 0)

        idx_load(0, 0)
        gathers_start(0)

        def pipe_body(j, _):
            for u in range(2):
                c = 2 * j + u
                p, q = u, 1 - u

                @pl.when(c >= 1)
                def _():
                    scat_wait(q)

                @pl.when(c + 1 < NFULL)
                def _():
                    idx_load(c + 1, q)
                    gathers_start(q)

                gathers_wait(p)
                compute(p, K)
                scat_start(p)
            return 0
        lax.fori_loop(0, NFULL // 2, pipe_body, 0)
        scat_wait((NFULL - 1) % 2)

        if TAIL:
            tb = ebase + NFULL * K
            pltpu.sync_copy(src_hbm.at[pl.ds(tb, TAIL)], ixs_t)
            pltpu.sync_copy(dst_hbm.at[pl.ds(tb, TAIL)], ixd_t)
            pltpu.async_copy(a8_hbm.at[ixs_t], ra0.at[pl.ds(0, TAIL)],
                             gsem0).wait()
            pltpu.async_copy(a8_hbm.at[ixd_t], rb0.at[pl.ds(0, TAIL)],
                             gsem0).wait()
            pltpu.async_copy(h_hbm.at[ixs_t], rows0.at[pl.ds(0, TAIL)],
                             gsem0).wait()
            compute(0, TAIL)
            pltpu.sync_copy(rows0.at[pl.ds(0, TAIL)], macc.at[ixd_t],
                            add=True)
            pltpu.sync_copy(wrow0.at[pl.ds(0, TAIL)], dacc.at[ixd_t],
                            add=True)

        plsc.subcore_barrier()
        pltpu.sync_copy(macc.at[pl.ds(rbase, RPT)],
                        msg_hbm.at[cid, pl.ds(rbase, RPT)])
        pltpu.sync_copy(dacc.at[pl.ds(rbase, RPT)],
                        den_hbm.at[cid, pl.ds(rbase, RPT)])

    return k(h, a8, src, dst)


def _tc_post(msg2, den2, batch3, W2, b2):
    blk = 1000
    nb = N // blk

    def body(m0_ref, m1_ref, d0_ref, d1_ref, b_ref, w2_ref, b2_ref,
             out_ref, gsum, gcnt):
        i = pl.program_id(0)

        @pl.when(i == 0)
        def _():
            gsum[...] = jnp.zeros((G, D), jnp.float32)
            gcnt[...] = jnp.zeros((G, D), jnp.float32)

        msg = m0_ref[0] + m1_ref[0]
        dsum = d0_ref[0] + d1_ref[0]
        den = dsum[:, 0:H] + jnp.float32(1e-16)
        expand = (lax.broadcasted_iota(jnp.int32, (H, D), 1) // DH ==
                  lax.broadcasted_iota(jnp.int32, (H, D), 0)
                  ).astype(jnp.float32)
        denf = jnp.dot(1.0 / den, expand, preferred_element_type=jnp.float32)
        node = msg * denf
        node = jnp.where(node > 0.0, node,
                         jnp.exp(jnp.minimum(node, 0.0)) - 1.0)
        bids = b_ref[0]
        gid = lax.broadcasted_iota(jnp.int32, (G, blk), 0)
        oh = (bids == gid).astype(jnp.float32)
        gsum[...] += jnp.dot(oh, node, preferred_element_type=jnp.float32)
        gcnt[...] += jnp.dot(oh, jnp.ones((blk, D), jnp.float32),
                             preferred_element_type=jnp.float32)

        @pl.when(i == nb - 1)
        def _():
            gr = gsum[...] / jnp.maximum(gcnt[...], 1.0)
            out_ref[...] = (jnp.dot(gr, w2_ref[...],
                                    preferred_element_type=jnp.float32)
                            + b2_ref[...])

    return pl.pallas_call(
        body,
        grid=(nb,),
        in_specs=[
            pl.BlockSpec((1, blk, D), lambda i: (0, i, 0)),
            pl.BlockSpec((1, blk, D), lambda i: (1, i, 0)),
            pl.BlockSpec((1, blk, 16), lambda i: (0, i, 0)),
            pl.BlockSpec((1, blk, 16), lambda i: (1, i, 0)),
            pl.BlockSpec((1, 1, blk), lambda i: (i, 0, 0)),
            pl.BlockSpec((D, C), lambda i: (0, 0)),
            pl.BlockSpec((1, C), lambda i: (0, 0)),
        ],
        out_specs=pl.BlockSpec((G, C), lambda i: (0, 0)),
        out_shape=jax.ShapeDtypeStruct((G, C), jnp.float32),
        scratch_shapes=[
            pltpu.VMEM((G, D), jnp.float32),
            pltpu.VMEM((G, D), jnp.float32),
        ],
    )(msg2, msg2, den2, den2, batch3, W2, b2)


def kernel(x, edge_index, batch, W, a_src, a_dst, W2, b2):
    Am = jnp.zeros((D, 2 * H), jnp.float32)
    for hd in range(H):
        Am = Am.at[hd * DH:(hd + 1) * DH, hd].set(a_src[hd])
        Am = Am.at[hd * DH:(hd + 1) * DH, H + hd].set(a_dst[hd])

    h, a8 = _tc_prep(x, W, Am)
    src = edge_index[0]
    dst = edge_index[1]

    msg, den = _sc_edges(h, a8, src, dst)

    batch3 = batch.reshape(N // 1000, 1, 1000)
    return _tc_post(msg, den, batch3, W2, b2.reshape(1, C))


# speedup vs baseline: 102.4694x; 1.3562x over previous
import functools

import jax
import jax.numpy as jnp
from jax import lax
from jax.experimental import pallas as pl
from jax.experimental.pallas import tpu as pltpu
from jax.experimental.pallas import tpu_sc as plsc

N = 10000
E = 320000
D = 128
H = 4
DH = 32
G = 64
C = 10

K = 96
NW = 32
EPW = E // NW
NFULL = EPW // K
TAIL = EPW - NFULL * K
N_PAD = 10240
RPT = N_PAD // 16


def _tc_prep(x, W, Am):
    blk = 1000
    nb = N // blk

    def body(x_ref, w_ref, am_ref, h_ref, a_ref):
        hb = jnp.dot(x_ref[...], w_ref[...], preferred_element_type=jnp.float32)
        h_ref[...] = hb
        a_ref[...] = jnp.dot(hb, am_ref[...], preferred_element_type=jnp.float32)

    return pl.pallas_call(
        body,
        grid=(nb,),
        in_specs=[
            pl.BlockSpec((blk, D), lambda i: (i, 0)),
            pl.BlockSpec((D, D), lambda i: (0, 0)),
            pl.BlockSpec((D, 8), lambda i: (0, 0)),
        ],
        out_specs=[
            pl.BlockSpec((blk, D), lambda i: (i, 0)),
            pl.BlockSpec((blk, 8), lambda i: (i, 0)),
        ],
        out_shape=[
            jax.ShapeDtypeStruct((N, D), jnp.float32),
            jax.ShapeDtypeStruct((N, 8), jnp.float32),
        ],
    )(x, W, Am)


def _sc_edges(h, a8, src, dst):
    mesh = plsc.VectorSubcoreMesh(core_axis_name="c", subcore_axis_name="s")

    @functools.partial(
        pl.kernel,
        mesh=mesh,
        out_type=[
            jax.ShapeDtypeStruct((2, N_PAD, D), jnp.float32),
            jax.ShapeDtypeStruct((2, N_PAD, 16), jnp.float32),
        ],
        compiler_params=pltpu.CompilerParams(use_tc_tiling_on_sc=False,
                                             needs_layout_passes=False),
        scratch_types=[
            pltpu.VMEM((K,), jnp.int32),
            pltpu.VMEM((K,), jnp.int32),
            pltpu.VMEM((K,), jnp.int32),
            pltpu.VMEM((K,), jnp.int32),
            pltpu.VMEM((16,), jnp.int32),
            pltpu.VMEM((16,), jnp.int32),
            pltpu.VMEM((K, D), jnp.float32),
            pltpu.VMEM((K, D), jnp.float32),
            pltpu.VMEM((K, 8), jnp.float32),
            pltpu.VMEM((K, 8), jnp.float32),
            pltpu.VMEM((K, 8), jnp.float32),
            pltpu.VMEM((K, 8), jnp.float32),
            pltpu.VMEM((K, 16), jnp.float32),
            pltpu.VMEM((K, 16), jnp.float32),
            pltpu.VMEM_SHARED((N_PAD, D), jnp.float32),
            pltpu.VMEM_SHARED((N_PAD, 16), jnp.float32),
            pltpu.SemaphoreType.DMA,
            pltpu.SemaphoreType.DMA,
            pltpu.SemaphoreType.DMA,
            pltpu.SemaphoreType.DMA,
        ],
    )
    def k(h_hbm, a8_hbm, src_hbm, dst_hbm, msg_hbm, den_hbm,
          ixs0, ixs1, ixd0, ixd1, ixs_t, ixd_t, rows0, rows1,
          ra0, ra1, rb0, rb1, wrow0, wrow1,
          macc, dacc, gsem0, gsem1, ssem0, ssem1):
        ixs = (ixs0, ixs1)
        ixd = (ixd0, ixd1)
        rows = (rows0, rows1)
        ra = (ra0, ra1)
        rb = (rb0, rb1)
        wrow = (wrow0, wrow1)
        gsem = (gsem0, gsem1)
        ssem = (ssem0, ssem1)

        cid = lax.axis_index("c")
        sid = lax.axis_index("s")
        wid = cid * 16 + sid
        ebase = wid * EPW
        lane = lax.iota(jnp.int32, 16)
        zv = jnp.zeros((16,), jnp.float32)

        def zrow(r, _):
            for c8 in range(D // 16):
                rows0[r, pl.ds(c8 * 16, 16)] = zv
            wrow0[r, pl.ds(0, 16)] = zv
            wrow1[r, pl.ds(0, 16)] = zv
            return 0
        lax.fori_loop(0, K, zrow, 0)
        rbase = sid * RPT
        nzf = RPT // K
        for j in range(nzf):
            pltpu.sync_copy(rows0, macc.at[pl.ds(rbase + j * K, K)])
            pltpu.sync_copy(wrow0, dacc.at[pl.ds(rbase + j * K, K)])
        zr = RPT - nzf * K
        if zr:
            pltpu.sync_copy(rows0.at[pl.ds(0, zr)],
                            macc.at[pl.ds(rbase + nzf * K, zr)])
            pltpu.sync_copy(wrow0.at[pl.ds(0, zr)],
                            dacc.at[pl.ds(rbase + nzf * K, zr)])
        plsc.subcore_barrier()

        def idx_load(c, p):
            base = ebase + c * K
            pltpu.sync_copy(src_hbm.at[pl.ds(base, K)], ixs[p])
            pltpu.sync_copy(dst_hbm.at[pl.ds(base, K)], ixd[p])

        def gathers_start(p):
            pltpu.async_copy(h_hbm.at[ixs[p]], rows[p], gsem[p])
            pltpu.async_copy(a8_hbm.at[ixs[p]], ra[p], gsem[p])
            pltpu.async_copy(a8_hbm.at[ixd[p]], rb[p], gsem[p])

        def gathers_wait(p):
            pltpu.make_async_copy(h_hbm.at[ixs[p]], rows[p], gsem[p]).wait()
            pltpu.make_async_copy(a8_hbm.at[ixs[p]], ra[p], gsem[p]).wait()
            pltpu.make_async_copy(a8_hbm.at[ixd[p]], rb[p], gsem[p]).wait()

        def scat_start(p):
            pltpu.async_copy(rows[p], macc.at[ixd[p]], ssem[p], add=True)
            pltpu.async_copy(wrow[p], dacc.at[ixd[p]], ssem[p], add=True)

        def scat_wait(p):
            pltpu.make_async_copy(rows[p], macc.at[ixd[p]], ssem[p]).wait()
            pltpu.make_async_copy(wrow[p], dacc.at[ixd[p]], ssem[p]).wait()

        lane4 = lax.rem(lane, jnp.int32(H))

        def compute(p, nk):
            for g in range(nk // 16):
                rix = lane + (g * 16)
                for hd in range(H):
                    e = (plsc.load_gather(ra[p], [rix, jnp.full((16,), hd,
                                                                jnp.int32)]) +
                         plsc.load_gather(rb[p], [rix, jnp.full((16,), 4 + hd,
                                                                jnp.int32)]))
                    e = jnp.where(e >= 0.0, e, e * jnp.float32(0.2))
                    w = jnp.exp(e)
                    plsc.store_scatter(
                        wrow[p], [rix, jnp.full((16,), hd, jnp.int32)], w)
            def edge_body(e2, _):
                wb = plsc.load_gather(
                    wrow[p], [jnp.full((16,), e2, jnp.int32), lane4])
                for c8 in range(D // 16):
                    off = c8 * 16
                    rows[p][e2, pl.ds(off, 16)] = (
                        rows[p][e2, pl.ds(off, 16)] * wb)
                return 0
            lax.fori_loop(0, nk, edge_body, 0, unroll=4)

        idx_load(0, 0)
        gathers_start(0)

        def pipe_body(j, _):
            for u in range(2):
                c = 2 * j + u
                p, q = u, 1 - u

                @pl.when(c >= 1)
                def _():
                    scat_wait(q)

                @pl.when(c + 1 < NFULL)
                def _():
                    idx_load(c + 1, q)
                    gathers_start(q)

                gathers_wait(p)
                compute(p, K)
                scat_start(p)
            return 0
        lax.fori_loop(0, NFULL // 2, pipe_body, 0)
        scat_wait((NFULL - 1) % 2)

        if TAIL:
            tb = ebase + NFULL * K
            pltpu.sync_copy(src_hbm.at[pl.ds(tb, TAIL)], ixs_t)
            pltpu.sync_copy(dst_hbm.at[pl.ds(tb, TAIL)], ixd_t)
            pltpu.async_copy(a8_hbm.at[ixs_t], ra0.at[pl.ds(0, TAIL)],
                             gsem0).wait()
            pltpu.async_copy(a8_hbm.at[ixd_t], rb0.at[pl.ds(0, TAIL)],
                             gsem0).wait()
            pltpu.async_copy(h_hbm.at[ixs_t], rows0.at[pl.ds(0, TAIL)],
                             gsem0).wait()
            compute(0, TAIL)
            pltpu.sync_copy(rows0.at[pl.ds(0, TAIL)], macc.at[ixd_t],
                            add=True)
            pltpu.sync_copy(wrow0.at[pl.ds(0, TAIL)], dacc.at[ixd_t],
                            add=True)

        plsc.subcore_barrier()
        pltpu.sync_copy(macc.at[pl.ds(rbase, RPT)],
                        msg_hbm.at[cid, pl.ds(rbase, RPT)])
        pltpu.sync_copy(dacc.at[pl.ds(rbase, RPT)],
                        den_hbm.at[cid, pl.ds(rbase, RPT)])

    return k(h, a8, src, dst)


def _tc_post(msg2, den2, batch3, W2, b2):
    blk = 1000
    nb = N // blk

    def body(m0_ref, m1_ref, d0_ref, d1_ref, b_ref, w2_ref, b2_ref,
             out_ref, gsum, gcnt):
        i = pl.program_id(0)

        @pl.when(i == 0)
        def _():
            gsum[...] = jnp.zeros((G, D), jnp.float32)
            gcnt[...] = jnp.zeros((G, D), jnp.float32)

        msg = m0_ref[0] + m1_ref[0]
        dsum = d0_ref[0] + d1_ref[0]
        den = dsum[:, 0:H] + jnp.float32(1e-16)
        expand = (lax.rem(lax.broadcasted_iota(jnp.int32, (H, D), 1),
                          jnp.int32(H)) ==
                  lax.broadcasted_iota(jnp.int32, (H, D), 0)
                  ).astype(jnp.float32)
        denf = jnp.dot(1.0 / den, expand, preferred_element_type=jnp.float32)
        node = msg * denf
        node = jnp.where(node > 0.0, node,
                         jnp.exp(jnp.minimum(node, 0.0)) - 1.0)
        bids = b_ref[0]
        gid = lax.broadcasted_iota(jnp.int32, (G, blk), 0)
        oh = (bids == gid).astype(jnp.float32)
        gsum[...] += jnp.dot(oh, node, preferred_element_type=jnp.float32)
        gcnt[...] += jnp.dot(oh, jnp.ones((blk, D), jnp.float32),
                             preferred_element_type=jnp.float32)

        @pl.when(i == nb - 1)
        def _():
            gr = gsum[...] / jnp.maximum(gcnt[...], 1.0)
            out_ref[...] = (jnp.dot(gr, w2_ref[...],
                                    preferred_element_type=jnp.float32)
                            + b2_ref[...])

    return pl.pallas_call(
        body,
        grid=(nb,),
        in_specs=[
            pl.BlockSpec((1, blk, D), lambda i: (0, i, 0)),
            pl.BlockSpec((1, blk, D), lambda i: (1, i, 0)),
            pl.BlockSpec((1, blk, 16), lambda i: (0, i, 0)),
            pl.BlockSpec((1, blk, 16), lambda i: (1, i, 0)),
            pl.BlockSpec((1, 1, blk), lambda i: (i, 0, 0)),
            pl.BlockSpec((D, C), lambda i: (0, 0)),
            pl.BlockSpec((1, C), lambda i: (0, 0)),
        ],
        out_specs=pl.BlockSpec((G, C), lambda i: (0, 0)),
        out_shape=jax.ShapeDtypeStruct((G, C), jnp.float32),
        scratch_shapes=[
            pltpu.VMEM((G, D), jnp.float32),
            pltpu.VMEM((G, D), jnp.float32),
        ],
    )(msg2, msg2, den2, den2, batch3, W2, b2)


def kernel(x, edge_index, batch, W, a_src, a_dst, W2, b2):
    perm = jnp.asarray([(d % H) * DH + d // H for d in range(D)], jnp.int32)
    Am = jnp.zeros((D, 2 * H), jnp.float32)
    for hd in range(H):
        Am = Am.at[hd * DH:(hd + 1) * DH, hd].set(a_src[hd])
        Am = Am.at[hd * DH:(hd + 1) * DH, H + hd].set(a_dst[hd])
    W_p = W[:, perm]
    Am_p = Am[perm, :]
    W2_p = W2[perm, :]

    h, a8 = _tc_prep(x, W_p, Am_p)
    src = edge_index[0]
    dst = edge_index[1]

    msg, den = _sc_edges(h, a8, src, dst)

    batch3 = batch.reshape(N // 1000, 1, 1000)
    return _tc_post(msg, den, batch3, W2_p, b2.reshape(1, C))
